# Initial kernel scaffold; baseline (speedup 1.0000x reference)
import jax
import jax.numpy as jnp
from jax.experimental import pallas as pl


def kernel(input, enc_w_in, enc_cab_w1, enc_cab_w2, enc_ca_w1, enc_ca_w2, enc_w_out, embed, dec_w_in, dec_cab_w1, dec_cab_w2, dec_ca_w1, dec_ca_w2, dec_w_out, alpha):
    raise NotImplementedError("write your pallas kernel here")


# speedup vs baseline: 1.2207x; 1.2207x over previous
import functools

import jax
import jax.numpy as jnp
from jax.experimental import pallas as pl

_TILE = 6400
_TILE_CAB = 1600


def _valid_mask(flat, Wp, W):
    first = Wp + 1
    last = W * (Wp + 1)
    mod = flat % Wp
    return (flat >= first) & (flat <= last) & (mod >= 1) & (mod <= W)


def _split3(x):
    h = x.astype(jnp.bfloat16)
    l = (x - h.astype(jnp.float32)).astype(jnp.bfloat16)
    return h, l


def _mm3(x, w):
    return jnp.dot(x, w, preferred_element_type=jnp.float32)


def _conv1x1_kernel(x_ref, w_ref, o_ref):
    o_ref[...] = _mm3(x_ref[0], w_ref[...])[None]


def _cab_main_kernel(xp_ref, xc_ref, xn_ref, w1_ref, w2_ref, a_ref,
                     res2_ref, psum_ref, *, M, Wp, W):
    h1 = Wp + 1
    halo = 2 * h1
    i = pl.program_id(1)
    C = w1_ref.shape[1]

    xw = jnp.concatenate(
        [xp_ref[0, M - halo:, :], xc_ref[0], xn_ref[0, :halo, :]], axis=0)

    n1 = M + halo
    x9 = jnp.concatenate(
        [xw[h1 + (k // 3) * Wp + (k % 3) - h1:][:n1] for k in range(9)],
        axis=1)
    res = _mm3(x9, w1_ref[...])
    a = a_ref[0, 0]
    res = jnp.where(res >= 0, res, a * res)
    flat1 = jax.lax.broadcasted_iota(jnp.int32, (n1, 1), 0) + (i * M - h1)
    res = jnp.where(_valid_mask(flat1, Wp, W), res, 0.0)

    r9 = jnp.concatenate(
        [res[(k // 3) * Wp + (k % 3):][:M] for k in range(9)], axis=1)
    res2 = _mm3(r9, w2_ref[...])
    flat2 = jax.lax.broadcasted_iota(jnp.int32, (M, 1), 0) + i * M
    res2 = jnp.where(_valid_mask(flat2, Wp, W), res2, 0.0)

    res2_ref[...] = res2[None]
    psum_ref[...] = jnp.sum(res2, axis=0)[None, None, None, :]


def _cab_apply_kernel(res2_ref, x_ref, ps_ref, ca1_ref, ca2_ref, o_ref,
                      *, nvalid):
    s = jnp.sum(ps_ref[0], axis=(0, 1)) * (1.0 / nvalid)
    t = _mm3(s[None, :], ca1_ref[...])
    t = jnp.maximum(t, 0.0)
    y = _mm3(t, ca2_ref[...])
    y = 1.0 / (1.0 + jnp.exp(-y))
    o_ref[...] = (res2_ref[0] * y + x_ref[0])[None]


def _vq_kernel(z_ref, emb_ref, q_ref, dsq_ref, *, M, Wp, W):
    i = pl.program_id(1)
    z = z_ref[0]
    emb = emb_ref[...]
    K = emb.shape[1]
    z2 = jnp.sum(z * z, axis=1, keepdims=True)
    e2 = jnp.sum(emb * emb, axis=0, keepdims=True)
    dist = z2 - 2.0 * _mm3(z, emb) + e2
    dmin = jnp.min(dist, axis=1, keepdims=True)
    col = jax.lax.broadcasted_iota(jnp.int32, dist.shape, 1)
    idx = jnp.min(jnp.where(dist == dmin, col, K), axis=1, keepdims=True)
    onehot = (col == idx).astype(jnp.float32)
    quant = jax.lax.dot_general(onehot, emb, (((1,), (1,)), ((), ())),
                                preferred_element_type=jnp.float32)
    flat = jax.lax.broadcasted_iota(jnp.int32, (M, 1), 0) + i * M
    v = _valid_mask(flat, Wp, W)
    quant = jnp.where(v, quant, 0.0)
    q_ref[...] = quant[None]
    d = jnp.where(v, quant - z, 0.0)
    dsq_ref[...] = jnp.sum(d * d).reshape(1, 1, 1, 1)


def kernel(input, enc_w_in, enc_cab_w1, enc_cab_w2, enc_ca_w1, enc_ca_w2,
           enc_w_out, embed, dec_w_in, dec_cab_w1, dec_cab_w2, dec_ca_w1,
           dec_ca_w2, dec_w_out, alpha):
    f32 = jnp.float32
    B, Cin, H, W = input.shape
    HID = enc_w_in.shape[0]
    EMB = enc_w_out.shape[0]
    NCAB = enc_cab_w1.shape[0]
    Hp, Wp = H + 2, W + 2
    L0 = Hp * Wp
    halo = 2 * (Wp + 1)

    nT = max(1, -(-L0 // _TILE))
    M = ((-(-L0 // nT)) + 63) // 64 * 64
    L = M * nT
    Mv = max(64, M // 4)
    while L % Mv:
        Mv //= 2
    nTv = L // Mv
    Mc = max(64, min(M, _TILE_CAB))
    while L % Mc or Mc < halo:
        Mc *= 2
    nTc = L // Mc
    assert M >= halo and Mc >= halo

    x = jnp.transpose(input, (0, 2, 3, 1))
    x = jnp.pad(x, ((0, 0), (1, 1), (1, 1), (0, 0)))
    x = x.reshape(B, L0, Cin)
    x = jnp.pad(x, ((0, 0), (0, L - L0), (0, 0)))

    def mm_w(w):
        return jnp.transpose(w[:, :, 0, 0], (1, 0))

    def cab_w(w):
        return jnp.transpose(w, (0, 3, 4, 2, 1)).reshape(NCAB,
                                                         9 * w.shape[2],
                                                         w.shape[1])

    enc_win = mm_w(enc_w_in)
    enc_w1 = cab_w(enc_cab_w1)
    enc_w2 = cab_w(enc_cab_w2)
    enc_a1 = jnp.transpose(enc_ca_w1[:, :, :, 0, 0], (0, 2, 1))
    enc_a2 = jnp.transpose(enc_ca_w2[:, :, :, 0, 0], (0, 2, 1))
    enc_wout = mm_w(enc_w_out)
    dec_win = mm_w(dec_w_in)
    dec_w1 = cab_w(dec_cab_w1)
    dec_w2 = cab_w(dec_cab_w2)
    dec_a1 = jnp.transpose(dec_ca_w1[:, :, :, 0, 0], (0, 2, 1))
    dec_a2 = jnp.transpose(dec_ca_w2[:, :, :, 0, 0], (0, 2, 1))
    dec_wout = mm_w(dec_w_out)
    a2d = jnp.reshape(alpha.astype(f32), (1, 1))

    def conv1x1(xx, w):
        Ci, Co = w.shape
        return pl.pallas_call(
            _conv1x1_kernel,
            grid=(B, nT),
            in_specs=[pl.BlockSpec((1, M, Ci), lambda b, i: (b, i, 0)),
                      pl.BlockSpec((Ci, Co), lambda b, i: (0, 0))],
            out_specs=pl.BlockSpec((1, M, Co), lambda b, i: (b, i, 0)),
            out_shape=jax.ShapeDtypeStruct((B, L, Co), f32),
        )(xx, w)

    def cab_main(xx, w1, w2):
        f = functools.partial(_cab_main_kernel, M=Mc, Wp=Wp, W=W)
        return pl.pallas_call(
            f,
            grid=(B, nTc),
            in_specs=[
                pl.BlockSpec((1, Mc, HID),
                             lambda b, i: (b, jnp.maximum(i - 1, 0), 0)),
                pl.BlockSpec((1, Mc, HID), lambda b, i: (b, i, 0)),
                pl.BlockSpec((1, Mc, HID),
                             lambda b, i: (b, jnp.minimum(i + 1, nTc - 1), 0)),
                pl.BlockSpec((9 * HID, HID), lambda b, i: (0, 0)),
                pl.BlockSpec((9 * HID, HID), lambda b, i: (0, 0)),
                pl.BlockSpec((1, 1), lambda b, i: (0, 0)),
            ],
            out_specs=[
                pl.BlockSpec((1, Mc, HID), lambda b, i: (b, i, 0)),
                pl.BlockSpec((1, 1, 1, HID), lambda b, i: (b, i, 0, 0)),
            ],
            out_shape=[jax.ShapeDtypeStruct((B, L, HID), f32),
                       jax.ShapeDtypeStruct((B, nTc, 1, HID), f32)],
        )(xx, xx, xx, w1, w2, a2d)

    def cab_apply(res2, xx, ps, ca1, ca2):
        RED = ca1.shape[1]
        f = functools.partial(_cab_apply_kernel, nvalid=float(H * W))
        return pl.pallas_call(
            f,
            grid=(B, nT),
            in_specs=[
                pl.BlockSpec((1, M, HID), lambda b, i: (b, i, 0)),
                pl.BlockSpec((1, M, HID), lambda b, i: (b, i, 0)),
                pl.BlockSpec((1, nTc, 1, HID), lambda b, i: (b, 0, 0, 0)),
                pl.BlockSpec((HID, RED), lambda b, i: (0, 0)),
                pl.BlockSpec((RED, HID), lambda b, i: (0, 0)),
            ],
            out_specs=pl.BlockSpec((1, M, HID), lambda b, i: (b, i, 0)),
            out_shape=jax.ShapeDtypeStruct((B, L, HID), f32),
        )(res2, xx, ps, ca1, ca2)

    def vq(zz):
        Kc = embed.shape[1]
        f = functools.partial(_vq_kernel, M=Mv, Wp=Wp, W=W)
        return pl.pallas_call(
            f,
            grid=(B, nTv),
            in_specs=[pl.BlockSpec((1, Mv, EMB), lambda b, i: (b, i, 0)),
                      pl.BlockSpec((EMB, Kc), lambda b, i: (0, 0))],
            out_specs=[
                pl.BlockSpec((1, Mv, EMB), lambda b, i: (b, i, 0)),
                pl.BlockSpec((1, 1, 1, 1), lambda b, i: (b, i, 0, 0)),
            ],
            out_shape=[jax.ShapeDtypeStruct((B, L, EMB), f32),
                       jax.ShapeDtypeStruct((B, nTv, 1, 1), f32)],
        )(zz, embed)

    h = conv1x1(x, enc_win)
    for c in range(NCAB):
        res2, ps = cab_main(h, enc_w1[c], enc_w2[c])
        h = cab_apply(res2, h, ps, enc_a1[c], enc_a2[c])
    z = conv1x1(h, enc_wout)

    quant, dsq = vq(z)
    diff = jnp.reshape(jnp.sum(dsq) / (B * H * W * EMB), (1,))

    q = conv1x1(quant, dec_win)
    for c in range(NCAB):
        res2, ps = cab_main(q, dec_w1[c], dec_w2[c])
        q = cab_apply(res2, q, ps, dec_a1[c], dec_a2[c])
    q = conv1x1(q, dec_wout)

    out = q[:, :L0, 0].reshape(B, Hp, Wp)[:, 1:H + 1, 1:W + 1]
    return out[:, None, :, :], diff
